# baseline (speedup 1.0000x reference)
import functools

import numpy as np
import jax
import jax.numpy as jnp
from jax import lax
from jax.experimental import pallas as pl
from jax.experimental.pallas import tpu as pltpu
from jax.experimental.pallas import tpu_sc as plsc

_VOCAB = 1000000
_EMB = 64
_SEQ = 2048
_BATCH = 4
_NTOK = _BATCH * _SEQ

_NC = 2
_NS = 16
_NW = _NC * _NS
_BPW = _NTOK // _NW
_LANES = 16
_CHUNKS = _EMB // _LANES


def _pe_host(seq: int, d: int) -> np.ndarray:
    pos = np.arange(seq, dtype=np.float64)[:, None]
    index = np.arange(d, dtype=np.float64)[None, :]
    tmp = pos / np.power(10000.0, index / float(d))
    pe = np.zeros((seq, d), dtype=np.float64)
    pe[:, 0::2] = np.sin(tmp[:, 0::2])
    pe[:, 1::2] = np.cos(tmp[:, 1::2])
    return pe.astype(np.float32)


_PE = _pe_host(_SEQ, _EMB)

_mesh = plsc.VectorSubcoreMesh(core_axis_name="c", subcore_axis_name="s")


@functools.partial(
    pl.kernel,
    mesh=_mesh,
    out_type=jax.ShapeDtypeStruct((_NTOK, _EMB), jnp.float32),
    scratch_types=[
        pltpu.VMEM((_BPW,), jnp.int32),
        pltpu.VMEM((_BPW, _EMB), jnp.float32),
        pltpu.VMEM((_BPW, _EMB), jnp.float32),
        pltpu.SemaphoreType.DMA,
    ],
    compiler_params=pltpu.CompilerParams(use_tc_tiling_on_sc=False),
)
def _embed_sc(table_hbm, idx_hbm, pe_hbm, out_hbm, idx_v, rows_v, pe_v, sem):
    wid = lax.axis_index("s") * _NC + lax.axis_index("c")
    base = wid * _BPW
    pe_base = lax.rem(base, _SEQ)

    pltpu.sync_copy(idx_hbm.at[pl.ds(base, _BPW)], idx_v)
    cps = [
        pltpu.async_copy(
            table_hbm.at[idx_v.at[pl.ds(j * 128, 128)]],
            rows_v.at[pl.ds(j * 128, 128)],
            sem,
        )
        for j in range(_BPW // 128)
    ]
    pltpu.sync_copy(pe_hbm.at[pl.ds(pe_base, _BPW)], pe_v)
    for cp in cps:
        cp.wait()

    def group_body(g, carry):
        idxv = idx_v[pl.ds(g * _LANES, _LANES)]
        for r in range(_LANES):
            row = g * _LANES + r
            m = jnp.where(idxv[r] != 0, 1.0, 0.0)
            for c in range(_CHUNKS):
                sl = pl.ds(c * _LANES, _LANES)
                rows_v[row, sl] = rows_v[row, sl] * m + pe_v[row, sl]
        return carry

    lax.fori_loop(0, _BPW // _LANES, group_body, 0)

    pltpu.sync_copy(rows_v, out_hbm.at[pl.ds(base, _BPW)])


def kernel(input, table):
    idx_flat = input.reshape(_NTOK)
    pe = jnp.asarray(_PE)
    out = _embed_sc(table, idx_flat, pe)
    return out.reshape(_BATCH, _SEQ, _EMB)


# speedup vs baseline: 3.7448x; 3.7448x over previous
import functools

import numpy as np
import jax
import jax.numpy as jnp
from jax import lax
from jax.experimental import pallas as pl
from jax.experimental.pallas import tpu as pltpu
from jax.experimental.pallas import tpu_sc as plsc

_VOCAB = 1000000
_EMB = 64
_SEQ = 2048
_BATCH = 4
_NTOK = _BATCH * _SEQ

_NC = 2
_NS = 16
_NW = _NC * _NS
_LANES = 16

_WCOLS = 512
_NWIN = -(-_VOCAB // _WCOLS)
_WPT = -(-_NWIN // _NW)
_LAST_COL0 = -(-(_VOCAB - _WCOLS) // 128) * 128
_OUT_ROWS = _NTOK


def _pe_host(seq: int, d: int) -> np.ndarray:
    pos = np.arange(seq, dtype=np.float64)[:, None]
    index = np.arange(d, dtype=np.float64)[None, :]
    tmp = pos / np.power(10000.0, index / float(d))
    pe = np.zeros((seq, d), dtype=np.float64)
    pe[:, 0::2] = np.sin(tmp[:, 0::2])
    pe[:, 1::2] = np.cos(tmp[:, 1::2])
    return pe.astype(np.float32)


_PE = _pe_host(_SEQ, _EMB)

_mesh = plsc.VectorSubcoreMesh(core_axis_name="c", subcore_axis_name="s")


@functools.partial(
    pl.kernel,
    mesh=_mesh,
    compiler_params=pltpu.CompilerParams(needs_layout_passes=False),
    out_type=jax.ShapeDtypeStruct((_OUT_ROWS, 128), jnp.float32),
    scratch_types=[
        pltpu.VMEM((_NTOK,), jnp.int32),
        pltpu.VMEM((_NTOK,), jnp.int32),
        pltpu.VMEM((_NTOK,), jnp.int32),
        pltpu.VMEM((_NTOK,), jnp.int32),
        pltpu.VMEM((_NTOK,), jnp.int32),
        pltpu.VMEM((_EMB, _WCOLS), jnp.float32),
        pltpu.VMEM((_EMB, _WCOLS), jnp.float32),
        pltpu.VMEM((_LANES, 128), jnp.float32),
        pltpu.VMEM((1, _LANES), jnp.int32),
        pltpu.SemaphoreType.DMA,
        pltpu.SemaphoreType.DMA,
        pltpu.SemaphoreType.DMA,
    ],
)
def _gather_sc(tablet_hbm, idx_hbm, out_hbm, idx_v, vlist, tlist, wcol, wtok,
               win0, win1, staging, srow, sem0, sem1, sem2):
    wid = lax.axis_index("s") * _NC + lax.axis_index("c")

    pltpu.sync_copy(idx_hbm, idx_v)

    lane = lax.broadcasted_iota(jnp.int32, (_LANES,), 0)

    def bucket_body(c, cnt):
        v = idx_v[pl.ds(c * _LANES, _LANES)]
        m = ((v >> 9) & (_NW - 1)) == wid
        plsc.store_compressed(vlist.at[pl.ds(cnt, _LANES)], v, mask=m)
        plsc.store_compressed(
            tlist.at[pl.ds(cnt, _LANES)], c * _LANES + lane, mask=m
        )
        npop = plsc.all_reduce_population_count(m)
        return cnt + npop[0]

    cnt = lax.fori_loop(0, _NTOK // _LANES, bucket_body, 0)
    nchunks = (cnt + _LANES - 1) // _LANES

    def fire(k, win, sem):
        w_glob = wid + _NW * k

        @pl.when(w_glob < _NWIN)
        def _():
            col0 = jnp.minimum(w_glob * _WCOLS, _LAST_COL0)
            col0 = pl.multiple_of(col0, 128)
            pltpu.async_copy(
                tablet_hbm.at[:, pl.ds(col0, _WCOLS)], win, sem
            )

    def process(k, win, sem):
        w_glob = wid + _NW * k

        @pl.when(w_glob < _NWIN)
        def _():
            pltpu.make_async_copy(
                tablet_hbm.at[:, pl.ds(0, _WCOLS)], win, sem
            ).wait()
            dma_col0 = jnp.minimum(w_glob * _WCOLS, _LAST_COL0)

            def sub_body(c, nw):
                valid = (c * _LANES + lane) < cnt
                v = vlist[pl.ds(c * _LANES, _LANES)]
                t = tlist[pl.ds(c * _LANES, _LANES)]
                m = valid & ((v >> 9) == w_glob)
                plsc.store_compressed(wcol.at[pl.ds(nw, _LANES)], v, mask=m)
                plsc.store_compressed(wtok.at[pl.ds(nw, _LANES)], t, mask=m)
                npop = plsc.all_reduce_population_count(m)
                return nw + npop[0]

            nw = lax.fori_loop(0, nchunks, sub_body, 0)

            def batch_body(b, carry):
                msk = (b * _LANES + lane) < nw
                jv = wcol[pl.ds(b * _LANES, _LANES)] - dma_col0
                tv = wtok[pl.ds(b * _LANES, _LANES)]
                jv = jnp.where(msk, jv, jv[0])
                tv = jnp.where(msk, tv, tv[0])
                for e in range(_EMB):
                    g = plsc.load_gather(
                        win, [jnp.full((_LANES,), e, jnp.int32), jv]
                    )
                    plsc.store_scatter(
                        staging,
                        [lane, jnp.full((_LANES,), e, jnp.int32)],
                        g,
                    )
                srow[0, :] = tv
                cp = pltpu.async_copy(staging, out_hbm.at[srow.at[0]], sem2)
                cp.wait()
                return carry

            lax.fori_loop(0, (nw + _LANES - 1) // _LANES, batch_body, 0)

    fire(0, win0, sem0)

    def step(i, carry):
        fire(2 * i + 1, win1, sem1)
        process(2 * i, win0, sem0)
        fire(2 * i + 2, win0, sem0)
        process(2 * i + 1, win1, sem1)
        return carry

    lax.fori_loop(0, _WPT // 2, step, 0)


def _epilogue_body(raw_ref, idx_ref, pe_ref, out_ref):
    rows = raw_ref[:, :_EMB]
    m = (idx_ref[0, 0, :] != 0).astype(jnp.float32).reshape(-1, 1)
    out_ref[...] = rows * m + pe_ref[...]


_EPI_BLK = 256


def _epilogue(raw, idx3, pe):
    return pl.pallas_call(
        _epilogue_body,
        grid=(_NTOK // _EPI_BLK,),
        in_specs=[
            pl.BlockSpec((_EPI_BLK, 128), lambda b: (b, 0)),
            pl.BlockSpec((1, 1, _EPI_BLK), lambda b: (b, 0, 0)),
            pl.BlockSpec((_EPI_BLK, _EMB), lambda b: (b % (_SEQ // _EPI_BLK), 0)),
        ],
        out_specs=pl.BlockSpec((_EPI_BLK, _EMB), lambda b: (b, 0)),
        out_shape=jax.ShapeDtypeStruct((_NTOK, _EMB), jnp.float32),
    )(raw, idx3, pe)


def kernel(input, table):
    idx_flat = input.reshape(_NTOK)
    raw = _gather_sc(table.T, idx_flat)
    idx3 = idx_flat.reshape(_NTOK // _EPI_BLK, 1, _EPI_BLK)
    pe = jnp.asarray(_PE)
    out = _epilogue(raw, idx3, pe)
    return out.reshape(_BATCH, _SEQ, _EMB)
